# baseline (speedup 1.0000x reference)
import functools

import jax
import jax.numpy as jnp
from jax import lax
from jax.experimental import pallas as pl
from jax.experimental.pallas import tpu as pltpu
from jax.experimental.pallas import tpu_sc as plsc

G = 128
NCHAN = 5
CHUNK = 128
NGROUPS = CHUNK // 16


def _sc_sample(posf, table, n_points):
    info = plsc.get_sparse_core_info()
    nc, ns = info.num_cores, info.num_subcores
    nw = nc * ns
    per_tile = n_points // nw
    n_chunks = per_tile // CHUNK

    mesh = plsc.VectorSubcoreMesh(core_axis_name="c", subcore_axis_name="s")

    @functools.partial(
        pl.kernel,
        mesh=mesh,
        compiler_params=pltpu.CompilerParams(
            needs_layout_passes=False, use_tc_tiling_on_sc=False),
        out_type=[
            jax.ShapeDtypeStruct((n_points,), jnp.float32),
            jax.ShapeDtypeStruct((n_points * 4,), jnp.float32),
        ],
        scratch_types=[
            pltpu.VMEM((3, CHUNK), jnp.float32),
            pltpu.VMEM((4, CHUNK), jnp.int32),
            pltpu.VMEM((4, CHUNK, 16), jnp.float32),
            pltpu.VMEM((CHUNK,), jnp.float32),
            pltpu.VMEM((CHUNK * 4,), jnp.float32),
            pltpu.SemaphoreType.DMA,
        ],
    )
    def k(px_hbm, py_hbm, pz_hbm, table_hbm, outh_hbm, oute_hbm,
          posbuf, idxbuf, gbuf, obufh, obufe, sem):
        wid = lax.axis_index("s") * nc + lax.axis_index("c")
        tile_base = wid * per_tile
        lane = lax.iota(jnp.int32, 16)

        def compute_xyz(p):
            px = posbuf[0, pl.ds(p, 16)]
            py = posbuf[1, pl.ds(p, 16)]
            pz = posbuf[2, pl.ds(p, 16)]
            x = jnp.clip((px + 1.0) * (0.5 * (G - 1)), 0.0, float(G - 1))
            y = jnp.clip((py + 1.0) * (0.5 * (G - 1)), 0.0, float(G - 1))
            z = jnp.clip((pz + 1.0) * (0.5 * (G - 1)), 0.0, float(G - 1))
            return x, y, z

        def phase1(g, carry):
            p = g * 16
            x, y, z = compute_xyz(p)
            xi = x.astype(jnp.int32)
            yi = y.astype(jnp.int32)
            zi = z.astype(jnp.int32)
            y1 = jnp.minimum(yi + 1, G - 1)
            z1 = jnp.minimum(zi + 1, G - 1)
            idxbuf[0, pl.ds(p, 16)] = (zi * G + yi) * G + xi
            idxbuf[1, pl.ds(p, 16)] = (zi * G + y1) * G + xi
            idxbuf[2, pl.ds(p, 16)] = (z1 * G + yi) * G + xi
            idxbuf[3, pl.ds(p, 16)] = (z1 * G + y1) * G + xi
            return carry

        def phase2(g, carry):
            p = g * 16
            x, y, z = compute_xyz(p)
            wx = x - x.astype(jnp.int32).astype(jnp.float32)
            wy = y - y.astype(jnp.int32).astype(jnp.float32)
            wz = z - z.astype(jnp.int32).astype(jnp.float32)
            ux = 1.0 - wx
            uy = 1.0 - wy
            uz = 1.0 - wz
            wk = (uz * uy, uz * wy, wz * uy, wz * wy)
            rowv = p + lane
            for c in range(NCHAN):
                acc = None
                for kk in range(4):
                    kv = jnp.full((16,), kk, jnp.int32)
                    v0 = plsc.load_gather(
                        gbuf, [kv, rowv, jnp.full((16,), c, jnp.int32)])
                    v1 = plsc.load_gather(
                        gbuf, [kv, rowv, jnp.full((16,), 8 + c, jnp.int32)])
                    term = wk[kk] * (v0 * ux + v1 * wx)
                    acc = term if acc is None else acc + term
                if c == 0:
                    obufh[pl.ds(p, 16)] = acc
                else:
                    plsc.store_scatter(obufe, [rowv * 4 + (c - 1)], acc)
            return carry

        def chunk_body(it, carry):
            base = tile_base + it * CHUNK
            for dd, ph in enumerate((px_hbm, py_hbm, pz_hbm)):
                pltpu.sync_copy(ph.at[pl.ds(base, CHUNK)], posbuf.at[dd])
            lax.fori_loop(0, NGROUPS, phase1, 0)
            cps = [pltpu.async_copy(table_hbm.at[idxbuf.at[kk]],
                                    gbuf.at[kk], sem)
                   for kk in range(4)]
            for cp in cps:
                cp.wait()
            lax.fori_loop(0, NGROUPS, phase2, 0)
            pltpu.sync_copy(obufh, outh_hbm.at[pl.ds(base, CHUNK)])
            pltpu.sync_copy(obufe, oute_hbm.at[pl.ds(base * 4, CHUNK * 4)])
            return carry

        lax.fori_loop(0, n_chunks, chunk_body, 0)

    return k(posf[0], posf[1], posf[2], table)


def kernel(positions, H, E):
    Bb, Tt, _ = positions.shape
    n_points = Bb * Tt
    vol = jnp.concatenate([H[0], E[0]], axis=0)
    vol = jnp.moveaxis(vol, 0, -1)
    vol8 = jnp.pad(vol, ((0, 0), (0, 0), (0, 0), (0, 3)))
    win1 = jnp.concatenate([vol8[:, :, 1:, :], vol8[:, :, G - 1:, :]], axis=2)
    table = jnp.concatenate([vol8, win1], axis=-1).reshape(G * G * G, 16)
    posf = positions.reshape(n_points, 3).T
    outh, oute = _sc_sample(posf, table, n_points)
    return outh.reshape(Bb, Tt), oute.reshape(Bb, Tt, 4)


# speedup vs baseline: 1.2854x; 1.2854x over previous
import functools

import jax
import jax.numpy as jnp
from jax import lax
from jax.experimental import pallas as pl
from jax.experimental.pallas import tpu as pltpu
from jax.experimental.pallas import tpu_sc as plsc

G = 128
NCHAN = 5
CHUNK = 128
NGROUPS = CHUNK // 16
NBUF = 2


def _sc_sample(posx, posy, posz, table, n_points):
    info = plsc.get_sparse_core_info()
    nc, ns = info.num_cores, info.num_subcores
    nw = nc * ns
    per_tile = n_points // nw
    n_chunks = per_tile // CHUNK

    mesh = plsc.VectorSubcoreMesh(core_axis_name="c", subcore_axis_name="s")

    @functools.partial(
        pl.kernel,
        mesh=mesh,
        compiler_params=pltpu.CompilerParams(
            needs_layout_passes=False, use_tc_tiling_on_sc=False),
        out_type=[
            jax.ShapeDtypeStruct((n_points,), jnp.float32),
            jax.ShapeDtypeStruct((n_points * 4,), jnp.float32),
        ],
        scratch_types=[
            pltpu.VMEM((NBUF, 3, CHUNK), jnp.float32),
            pltpu.VMEM((NBUF, 3, CHUNK), jnp.float32),
            pltpu.VMEM((NBUF, 4, CHUNK), jnp.int32),
            pltpu.VMEM((NBUF, 4, CHUNK, 16), jnp.float32),
            pltpu.VMEM((NBUF, CHUNK), jnp.float32),
            pltpu.VMEM((NBUF, CHUNK * 4), jnp.float32),
            pltpu.SemaphoreType.DMA,
            pltpu.SemaphoreType.DMA,
            pltpu.SemaphoreType.DMA,
        ],
    )
    def k(px_hbm, py_hbm, pz_hbm, table_hbm, outh_hbm, oute_hbm,
          posbuf, wbuf, idxbuf, gbuf, obufh, obufe, sem_pos, sem_g, sem_out):
        wid = lax.axis_index("s") * nc + lax.axis_index("c")
        tile_base = wid * per_tile
        lane = lax.iota(jnp.int32, 16)
        table2 = table_hbm

        def pos_copies(it):
            slot = lax.rem(it, NBUF)
            base = tile_base + it * CHUNK
            return [pltpu.make_async_copy(ph.at[pl.ds(base, CHUNK)],
                                          posbuf.at[slot, dd], sem_pos)
                    for dd, ph in enumerate((px_hbm, py_hbm, pz_hbm))]

        def gather_copies(it):
            slot = lax.rem(it, NBUF)
            return [pltpu.make_async_copy(table2.at[idxbuf.at[slot, kk]],
                                          gbuf.at[slot, kk], sem_g)
                    for kk in range(4)]

        def out_copies(it):
            slot = lax.rem(it, NBUF)
            base = tile_base + it * CHUNK
            return [
                pltpu.make_async_copy(obufh.at[slot],
                                      outh_hbm.at[pl.ds(base, CHUNK)],
                                      sem_out),
                pltpu.make_async_copy(obufe.at[slot],
                                      oute_hbm.at[pl.ds(base * 4, CHUNK * 4)],
                                      sem_out),
            ]

        def compute_xyz(slot, p):
            px = posbuf[slot, 0, pl.ds(p, 16)]
            py = posbuf[slot, 1, pl.ds(p, 16)]
            pz = posbuf[slot, 2, pl.ds(p, 16)]
            x = jnp.clip((px + 1.0) * (0.5 * (G - 1)), 0.0, float(G - 1))
            y = jnp.clip((py + 1.0) * (0.5 * (G - 1)), 0.0, float(G - 1))
            z = jnp.clip((pz + 1.0) * (0.5 * (G - 1)), 0.0, float(G - 1))
            return x, y, z

        def phase1(it):
            slot = lax.rem(it, NBUF)

            def body(g, carry):
                p = g * 16
                x, y, z = compute_xyz(slot, p)
                xi = x.astype(jnp.int32)
                yi = y.astype(jnp.int32)
                zi = z.astype(jnp.int32)
                y1 = jnp.minimum(yi + 1, G - 1)
                z1 = jnp.minimum(zi + 1, G - 1)
                idxbuf[slot, 0, pl.ds(p, 16)] = (zi * G + yi) * G + xi
                idxbuf[slot, 1, pl.ds(p, 16)] = (zi * G + y1) * G + xi
                idxbuf[slot, 2, pl.ds(p, 16)] = (z1 * G + yi) * G + xi
                idxbuf[slot, 3, pl.ds(p, 16)] = (z1 * G + y1) * G + xi
                wbuf[slot, 0, pl.ds(p, 16)] = x - xi.astype(jnp.float32)
                wbuf[slot, 1, pl.ds(p, 16)] = y - yi.astype(jnp.float32)
                wbuf[slot, 2, pl.ds(p, 16)] = z - zi.astype(jnp.float32)
                return carry

            lax.fori_loop(0, NGROUPS, body, 0)

        def phase2(it):
            slot = lax.rem(it, NBUF)

            def body(g, carry):
                p = g * 16
                wx = wbuf[slot, 0, pl.ds(p, 16)]
                wy = wbuf[slot, 1, pl.ds(p, 16)]
                wz = wbuf[slot, 2, pl.ds(p, 16)]
                ux = 1.0 - wx
                wk = ((1.0 - wz) * (1.0 - wy), (1.0 - wz) * wy,
                      wz * (1.0 - wy), wz * wy)
                rowv = p + lane
                slotv = jnp.full((16,), slot, jnp.int32)
                for c in range(NCHAN):
                    acc = None
                    for kk in range(4):
                        kv = jnp.full((16,), kk, jnp.int32)
                        v0 = plsc.load_gather(
                            gbuf,
                            [slotv, kv, rowv, jnp.full((16,), c, jnp.int32)])
                        v1 = plsc.load_gather(
                            gbuf,
                            [slotv, kv, rowv,
                             jnp.full((16,), 8 + c, jnp.int32)])
                        term = wk[kk] * (v0 * ux + v1 * wx)
                        acc = term if acc is None else acc + term
                    if c == 0:
                        obufh[slot, pl.ds(p, 16)] = acc
                    else:
                        plsc.store_scatter(obufe.at[slot],
                                           [rowv * 4 + (c - 1)], acc)
                return carry

            lax.fori_loop(0, NGROUPS, body, 0)

        for cp in pos_copies(0):
            cp.start()
        if n_chunks > 1:
            for cp in pos_copies(1):
                cp.start()

        def loop_body(it, carry):
            for cp in pos_copies(it):
                cp.wait()
            phase1(it)
            for cp in gather_copies(it):
                cp.start()

            @pl.when(it + 2 < n_chunks)
            def _():
                for cp in pos_copies(it + 2):
                    cp.start()

            @pl.when(it >= 1)
            def _():
                for cp in gather_copies(it - 1):
                    cp.wait()

                @pl.when(it >= 3)
                def _():
                    for cp in out_copies(it - 3):
                        cp.wait()

                phase2(it - 1)
                for cp in out_copies(it - 1):
                    cp.start()

            return carry

        lax.fori_loop(0, n_chunks, loop_body, 0)

        last = n_chunks - 1
        for cp in gather_copies(last):
            cp.wait()
        if n_chunks >= 3:
            for cp in out_copies(last - 2):
                cp.wait()
        phase2(last)
        for cp in out_copies(last):
            cp.start()
        if n_chunks >= 2:
            for cp in out_copies(last - 1):
                cp.wait()
        for cp in out_copies(last):
            cp.wait()

    return k(posx, posy, posz, table)


def kernel(positions, H, E):
    Bb, Tt, _ = positions.shape
    n_points = Bb * Tt
    vol = jnp.concatenate([H[0], E[0]], axis=0)
    vol = jnp.moveaxis(vol, 0, -1)
    vol8 = jnp.pad(vol, ((0, 0), (0, 0), (0, 0), (0, 3)))
    win1 = jnp.concatenate([vol8[:, :, 1:, :], vol8[:, :, G - 1:, :]], axis=2)
    table = jnp.concatenate([vol8, win1], axis=-1).reshape(G * G * G, 16)
    posf = positions.reshape(n_points, 3).T
    outh, oute = _sc_sample(posf[0], posf[1], posf[2], table, n_points)
    return outh.reshape(Bb, Tt), oute.reshape(Bb, Tt, 4)
